# baseline (speedup 1.0000x reference)
import functools

import jax
import jax.numpy as jnp
from jax import lax
from jax.experimental import pallas as pl
from jax.experimental.pallas import tpu as pltpu
from jax.experimental.pallas import tpu_sc as plsc

D_MODEL = 64
COL_PITCH = 1008
LANES = 16


@functools.lru_cache(maxsize=None)
def _build_sc_gather(seq: int, batch: int, vocab: int):
    info = plsc.get_sparse_core_info()
    num_workers = info.num_cores * info.num_subcores
    bsw = batch // num_workers
    assert bsw * num_workers == batch and bsw % LANES == 0
    n_groups = bsw // LANES

    mesh = plsc.VectorSubcoreMesh(core_axis_name="c", subcore_axis_name="s")

    @functools.partial(
        pl.kernel,
        mesh=mesh,
        out_type=jax.ShapeDtypeStruct((seq, D_MODEL, batch), jnp.float32),
        scratch_types=[
            pltpu.VMEM((D_MODEL * COL_PITCH,), jnp.float32),
            pltpu.VMEM((seq, bsw), jnp.int32),
            pltpu.VMEM((3, D_MODEL, bsw), jnp.float32),
            pltpu.SemaphoreType.DMA,
        ],
        compiler_params=pltpu.CompilerParams(
            use_tc_tiling_on_sc=False, needs_layout_passes=False),
    )
    def gather_kernel(table_hbm, idx_hbm, out_hbm, table_v, idx_v, slab_v, osem):
        wid = lax.axis_index("s") * info.num_cores + lax.axis_index("c")
        b0 = wid * bsw
        pltpu.sync_copy(table_hbm, table_v)
        pltpu.sync_copy(idx_hbm.at[:, pl.ds(b0, bsw)], idx_v)

        def compute(s, buf):
            for g in range(n_groups):
                base = idx_v[s, pl.ds(g * LANES, LANES)]
                prev = None
                for d0 in range(0, D_MODEL, 8):
                    cur = [plsc.load_gather(
                        table_v.at[pl.ds((d0 + k) * COL_PITCH, COL_PITCH)],
                        [base])
                        for k in range(8)]
                    if prev is not None:
                        for k in range(8):
                            slab_v[buf, d0 - 8 + k,
                                   pl.ds(g * LANES, LANES)] = prev[k]
                    prev = cur
                for k in range(8):
                    slab_v[buf, D_MODEL - 8 + k,
                           pl.ds(g * LANES, LANES)] = prev[k]

        def ship(s, buf):
            pltpu.async_copy(
                slab_v.at[buf], out_hbm.at[s, :, pl.ds(b0, bsw)], osem)

        def drain(buf):
            pltpu.make_async_copy(
                slab_v.at[buf], out_hbm.at[0, :, pl.ds(b0, bsw)], osem).wait()

        for s in range(3):
            compute(s, s)
            ship(s, s)

        def body(s, carry):
            buf = lax.rem(s, 3)
            drain(buf)
            compute(s, buf)
            ship(s, buf)
            return carry

        lax.fori_loop(3, seq, body, 0)
        for b in range(3):
            drain(b)

    return gather_kernel


def kernel(gene_pos, pe):
    b, s = gene_pos.shape
    vocab = pe.shape[0]
    idx_t = gene_pos.T
    table = jnp.pad(pe.reshape(vocab, D_MODEL).T,
                    ((0, 0), (0, COL_PITCH - vocab))).reshape(D_MODEL * COL_PITCH)
    out_t = _build_sc_gather(s, b, vocab)(table, idx_t)
    return jnp.transpose(out_t, (2, 0, 1))


# speedup vs baseline: 1.0193x; 1.0193x over previous
import functools

import jax
import jax.numpy as jnp
from jax import lax
from jax.experimental import pallas as pl
from jax.experimental.pallas import tpu as pltpu
from jax.experimental.pallas import tpu_sc as plsc

D_MODEL = 64
ROW_PITCH = 65
LANES = 16


@functools.lru_cache(maxsize=None)
def _build_sc_gather(seq: int, batch: int, vocab: int):
    info = plsc.get_sparse_core_info()
    num_workers = info.num_cores * info.num_subcores
    bsw = batch // num_workers
    assert bsw * num_workers == batch and bsw % LANES == 0
    n_groups = bsw // LANES

    mesh = plsc.VectorSubcoreMesh(core_axis_name="c", subcore_axis_name="s")

    @functools.partial(
        pl.kernel,
        mesh=mesh,
        out_type=jax.ShapeDtypeStruct((seq, D_MODEL, batch), jnp.float32),
        scratch_types=[
            pltpu.VMEM((vocab * ROW_PITCH,), jnp.float32),
            pltpu.VMEM((seq, bsw), jnp.int32),
            pltpu.VMEM((2, D_MODEL, bsw), jnp.float32),
            pltpu.SemaphoreType.DMA,
        ],
        compiler_params=pltpu.CompilerParams(
            use_tc_tiling_on_sc=False, needs_layout_passes=False),
    )
    def gather_kernel(table_hbm, idx_hbm, out_hbm, table_v, idx_v, slab_v, osem):
        wid = lax.axis_index("s") * info.num_cores + lax.axis_index("c")
        b0 = wid * bsw
        pltpu.sync_copy(table_hbm, table_v)
        pltpu.sync_copy(idx_hbm.at[:, pl.ds(b0, bsw)], idx_v)

        def compute(s, buf):
            for g in range(n_groups):
                base = idx_v[s, pl.ds(g * LANES, LANES)] * ROW_PITCH
                prev = None
                for d0 in range(0, D_MODEL, 16):
                    cur = [plsc.load_gather(table_v, [base + (d0 + k)])
                           for k in range(16)]
                    if prev is not None:
                        for k in range(16):
                            slab_v[buf, d0 - 16 + k,
                                   pl.ds(g * LANES, LANES)] = prev[k]
                    prev = cur
                for k in range(16):
                    slab_v[buf, D_MODEL - 16 + k,
                           pl.ds(g * LANES, LANES)] = prev[k]

        def ship(s, buf):
            pltpu.async_copy(
                slab_v.at[buf], out_hbm.at[s, :, pl.ds(b0, bsw)], osem)

        def drain(buf):
            pltpu.make_async_copy(
                slab_v.at[buf], out_hbm.at[0, :, pl.ds(b0, bsw)], osem).wait()

        for s in range(2):
            compute(s, s)
            ship(s, s)

        def body(s, carry):
            buf = lax.rem(s, 2)
            drain(buf)
            compute(s, buf)
            ship(s, buf)
            return carry

        lax.fori_loop(2, seq, body, 0)
        for b in range(2):
            drain(b)

    return gather_kernel


def kernel(gene_pos, pe):
    b, s = gene_pos.shape
    vocab = pe.shape[0]
    idx_t = gene_pos.T
    table = jnp.pad(pe.reshape(vocab, D_MODEL),
                    ((0, 0), (0, ROW_PITCH - D_MODEL))).reshape(vocab * ROW_PITCH)
    out_t = _build_sc_gather(s, b, vocab)(table, idx_t)
    return jnp.transpose(out_t, (2, 0, 1))


# speedup vs baseline: 1.0377x; 1.0181x over previous
import functools

import jax
import jax.numpy as jnp
from jax import lax
from jax.experimental import pallas as pl
from jax.experimental.pallas import tpu as pltpu
from jax.experimental.pallas import tpu_sc as plsc

D_MODEL = 64
ROW_PITCH = 65
LANES = 16


@functools.lru_cache(maxsize=None)
def _build_sc_gather(seq: int, batch: int, vocab: int):
    info = plsc.get_sparse_core_info()
    num_workers = info.num_cores * info.num_subcores
    bsw = batch // num_workers
    assert bsw * num_workers == batch and bsw % LANES == 0
    n_groups = bsw // LANES

    mesh = plsc.VectorSubcoreMesh(core_axis_name="c", subcore_axis_name="s")

    @functools.partial(
        pl.kernel,
        mesh=mesh,
        out_type=jax.ShapeDtypeStruct((seq, D_MODEL, batch), jnp.float32),
        scratch_types=[
            pltpu.VMEM((vocab * ROW_PITCH,), jnp.float32),
            pltpu.VMEM((seq, bsw), jnp.int32),
            pltpu.VMEM((2, D_MODEL, bsw), jnp.float32),
            pltpu.SemaphoreType.DMA,
        ],
        compiler_params=pltpu.CompilerParams(
            use_tc_tiling_on_sc=False, needs_layout_passes=False),
    )
    def gather_kernel(table_hbm, idx_hbm, out_hbm, table_v, idx_v, slab_v, osem):
        wid = lax.axis_index("s") * info.num_cores + lax.axis_index("c")
        b0 = wid * bsw
        pltpu.sync_copy(table_hbm, table_v)
        pltpu.sync_copy(idx_hbm.at[:, pl.ds(b0, bsw)], idx_v)

        def compute(s, buf):
            for g in range(n_groups):
                base = idx_v[s, pl.ds(g * LANES, LANES)] * ROW_PITCH
                prev = None
                for d0 in range(0, D_MODEL, 8):
                    cur = [plsc.load_gather(table_v, [base + (d0 + k)])
                           for k in range(8)]
                    if prev is not None:
                        for k in range(8):
                            slab_v[buf, d0 - 8 + k,
                                   pl.ds(g * LANES, LANES)] = prev[k]
                    prev = cur
                for k in range(8):
                    slab_v[buf, D_MODEL - 8 + k,
                           pl.ds(g * LANES, LANES)] = prev[k]

        def ship(s, buf):
            pltpu.async_copy(
                slab_v.at[buf], out_hbm.at[s, :, pl.ds(b0, bsw)], osem)

        def drain(buf):
            pltpu.make_async_copy(
                slab_v.at[buf], out_hbm.at[0, :, pl.ds(b0, bsw)], osem).wait()

        for s in range(2):
            compute(s, s)
            ship(s, s)

        def body(s, carry):
            buf = lax.rem(s, 2)
            drain(buf)
            compute(s, buf)
            ship(s, buf)
            return carry

        lax.fori_loop(2, seq, body, 0)
        for b in range(2):
            drain(b)

    return gather_kernel


def kernel(gene_pos, pe):
    b, s = gene_pos.shape
    vocab = pe.shape[0]
    idx_t = gene_pos.T
    table = jnp.pad(pe.reshape(vocab, D_MODEL),
                    ((0, 0), (0, ROW_PITCH - D_MODEL))).reshape(vocab * ROW_PITCH)
    out_t = _build_sc_gather(s, b, vocab)(table, idx_t)
    return jnp.transpose(out_t, (2, 0, 1))
